# baseline (speedup 1.0000x reference)
import functools

import jax
import jax.numpy as jnp
from jax import lax
from jax.experimental import pallas as pl
from jax.experimental.pallas import tpu as pltpu
from jax.experimental.pallas import tpu_sc as plsc

N = 10000
D = 128
E = 320000
NC, NS, L = 2, 16, 16
NW = NC * NS
EPW = E // NW
CH = 80
NCHUNK = EPW // CH
NP = 10240
RPT = NP // NS


def _norm_body(x_ref, o_ref):
    x = x_ref[...]
    ss = jnp.sum(x * x, axis=1, keepdims=True)
    nrm = jnp.maximum(jnp.sqrt(ss), 1e-12)
    o_ref[...] = x / nrm


def _normalize(raw):
    return pl.pallas_call(
        _norm_body,
        grid=(10,),
        in_specs=[pl.BlockSpec((1000, D), lambda i: (i, 0))],
        out_specs=pl.BlockSpec((1000, D), lambda i: (i, 0)),
        out_shape=jax.ShapeDtypeStruct((N, D), jnp.float32),
    )(raw)


_MESH = plsc.VectorSubcoreMesh(
    core_axis_name="c", subcore_axis_name="s", num_cores=NC, num_subcores=NS
)


@functools.partial(
    pl.kernel,
    out_type=jax.ShapeDtypeStruct((NC, NP, D), jnp.float32),
    mesh=_MESH,
    compiler_params=pltpu.CompilerParams(needs_layout_passes=False),
    scratch_types=[
        pltpu.VMEM((CH,), jnp.int32),
        pltpu.VMEM((CH,), jnp.int32),
        pltpu.VMEM((CH,), jnp.int32),
        pltpu.VMEM((CH,), jnp.int32),
        pltpu.VMEM((L,), jnp.int32),
        pltpu.VMEM((CH,), jnp.int32),
        pltpu.VMEM((CH,), jnp.int32),
        pltpu.VMEM((CH,), jnp.int32),
        pltpu.VMEM((CH,), jnp.int32),
        pltpu.VMEM((CH,), jnp.int32),
        pltpu.VMEM((CH,), jnp.int32),
        pltpu.VMEM((CH,), jnp.int32),
        pltpu.VMEM((CH,), jnp.int32),
        pltpu.VMEM((CH, D), jnp.float32),
        pltpu.VMEM((CH, D), jnp.float32),
        pltpu.VMEM((CH, D), jnp.float32),
        pltpu.VMEM((CH, D), jnp.float32),
        pltpu.VMEM_SHARED((NP, D), jnp.float32),
        pltpu.SemaphoreType.DMA,
        pltpu.SemaphoreType.DMA,
        pltpu.SemaphoreType.DMA,
        pltpu.SemaphoreType.DMA,
        pltpu.SemaphoreType.DMA,
        pltpu.SemaphoreType.DMA,
        pltpu.SemaphoreType.DMA,
        pltpu.SemaphoreType.DMA,
        pltpu.SemaphoreType.DMA,
        pltpu.SemaphoreType.DMA,
        pltpu.SemaphoreType.DMA,
        pltpu.SemaphoreType.DMA,
    ],
)
def _sc_agg(u_hbm, pk_hbm, et1_hbm, agg_hbm,
            pk0, pk1, pk2, pk3, tvec,
            src0, src1, src2, src3, dst0, dst1, dst2, dst3,
            rows0, rows1, rows2, rows3, acc_sh,
            sem0, sem1, sem2, sem3, ssem0, ssem1, ssem2, ssem3,
            psem0, psem1, psem2, psem3):
    c = lax.axis_index("c")
    s = lax.axis_index("s")
    wid = s * NC + c
    ebase = wid * EPW

    zero = jnp.zeros((L,), jnp.float32)

    def zrow(i, _):
        for j in range(D // L):
            rows0[i, pl.ds(j * L, L)] = zero
        return ()

    lax.fori_loop(0, CH, zrow, ())

    def zcopy(k, _):
        pltpu.sync_copy(rows0, acc_sh.at[pl.ds(s * RPT + k * CH, CH)])
        return ()

    lax.fori_loop(0, RPT // CH, zcopy, ())
    plsc.subcore_barrier()

    pltpu.sync_copy(et1_hbm.at[pl.ds(E - L, L)], tvec)
    at_vec = tvec[...].astype(jnp.float32) + 1.0
    inv_at = (1.0 / at_vec)[L - 1]

    bufs = (
        (rows0, src0, dst0, pk0, sem0, ssem0, psem0),
        (rows1, src1, dst1, pk1, sem1, ssem1, psem1),
        (rows2, src2, dst2, pk2, sem2, ssem2, psem2),
        (rows3, src3, dst3, pk3, sem3, ssem3, psem3),
    )
    NB = len(bufs)

    def load_pk(k, buf):
        pltpu.async_copy(pk_hbm.at[pl.ds(ebase + k * CH, CH)], buf[3], buf[6])

    def issue(k, rows_b, src_b, dst_b, pk_b, sem_b, ssem_b, psem_b):
        pltpu.make_async_copy(
            pk_hbm.at[pl.ds(ebase + k * CH, CH)], pk_b, psem_b).wait()
        for j in range(CH // L):
            p = pk_b[pl.ds(j * L, L)]
            src_b[pl.ds(j * L, L)] = p & 0x3FFF
        pltpu.async_copy(u_hbm.at[src_b], rows_b, sem_b)

    def drain_scatter(rows_b, src_b, dst_b, pk_b, sem_b, ssem_b, psem_b):
        pltpu.make_async_copy(rows_b, acc_sh.at[dst_b], ssem_b).wait()

    def step(k, b):
        rows_b, src_b, dst_b, pk_b, sem_b, ssem_b, psem_b = bufs[b]
        pltpu.make_async_copy(u_hbm.at[src_b], rows_b, sem_b).wait()
        for j in range(CH // L):
            p = pk_b[pl.ds(j * L, L)]
            dst_b[pl.ds(j * L, L)] = lax.shift_right_logical(p, 14) & 0x3FFF
            t16 = lax.shift_right_logical(p, 28)
            scale = (t16.astype(jnp.float32) + 1.0) * inv_at
            rid = lax.iota(jnp.int32, L) + (j * L)
            cid = jnp.full((L,), D - 1, jnp.int32)
            plsc.store_scatter(rows_b, (rid, cid), scale)

        pltpu.async_copy(rows_b, acc_sh.at[dst_b], ssem_b, add=True)

        @pl.when(k + 4 < NCHUNK)
        def _():
            load_pk(k + 4, bufs[b])

        nxt = bufs[(b + 2) % NB]

        @pl.when(k >= 2)
        def _():
            drain_scatter(*nxt)

        @pl.when(k + 2 < NCHUNK)
        def _():
            issue(k + 2, *nxt)

    for k in range(min(4, NCHUNK)):
        load_pk(k, bufs[k % NB])
    issue(0, *bufs[0])
    issue(1, *bufs[1])

    def quad(i, _):
        for b in range(NB):
            step(NB * i + b, b)
        return ()

    nfull = (NCHUNK - 2) // NB
    lax.fori_loop(0, nfull, quad, ())
    for k in range(NB * nfull, NCHUNK):
        step(k, k % NB)
    drain_scatter(*bufs[(NCHUNK - 2) % NB])
    drain_scatter(*bufs[(NCHUNK - 1) % NB])
    plsc.subcore_barrier()

    def ocopy(k, _):
        off = s * RPT + k * CH
        pltpu.sync_copy(acc_sh.at[pl.ds(off, CH)], agg_hbm.at[c, pl.ds(off, CH)])
        return ()

    lax.fori_loop(0, RPT // CH, ocopy, ())


def _fin_body(x_ref, a_ref, w_ref, o_ref):
    feat = x_ref[...] + a_ref[0] + a_ref[1]
    prod = lax.dot_general(
        feat, w_ref[...], (((1,), (1,)), ((), ())),
        preferred_element_type=jnp.float32,
    )
    o_ref[...] = jnp.tanh(prod)


def _finalize(raw, parts, W):
    return pl.pallas_call(
        _fin_body,
        grid=(10,),
        in_specs=[
            pl.BlockSpec((1000, D), lambda i: (i, 0)),
            pl.BlockSpec((NC, 1000, D), lambda i: (0, i, 0)),
            pl.BlockSpec((D, D), lambda i: (0, 0)),
        ],
        out_specs=pl.BlockSpec((1000, D), lambda i: (i, 0)),
        out_shape=jax.ShapeDtypeStruct((N, D), jnp.float32),
    )(raw, parts, W)


def kernel(raw_features, edge_index, edge_time, W):
    u = _normalize(raw_features)
    packed = edge_index[0] | (edge_index[1] << 14) | (edge_time << 28)
    parts = _sc_agg(u, packed, edge_time)
    return _finalize(raw_features, parts, W)


# speedup vs baseline: 1.0952x; 1.0952x over previous
import functools

import jax
import jax.numpy as jnp
from jax import lax
from jax.experimental import pallas as pl
from jax.experimental.pallas import tpu as pltpu
from jax.experimental.pallas import tpu_sc as plsc

N = 10000
D = 128
E = 320000
NC, NS, L = 2, 16, 16
NW = NC * NS
EPW = E // NW
CH = 80
NCHUNK = EPW // CH
NP = 10240
RPT = NP // NS


def _norm_body(x_ref, o_ref):
    x = x_ref[...]
    ss = jnp.sum(x * x, axis=1, keepdims=True)
    nrm = jnp.maximum(jnp.sqrt(ss), 1e-12)
    o_ref[...] = x / nrm


def _normalize(raw):
    return pl.pallas_call(
        _norm_body,
        grid=(10,),
        in_specs=[pl.BlockSpec((1000, D), lambda i: (i, 0))],
        out_specs=pl.BlockSpec((1000, D), lambda i: (i, 0)),
        out_shape=jax.ShapeDtypeStruct((N, D), jnp.float32),
    )(raw)


_MESH = plsc.VectorSubcoreMesh(
    core_axis_name="c", subcore_axis_name="s", num_cores=NC, num_subcores=NS
)


@functools.partial(
    pl.kernel,
    out_type=jax.ShapeDtypeStruct((NC, NP, D), jnp.float32),
    mesh=_MESH,
    compiler_params=pltpu.CompilerParams(needs_layout_passes=False),
    scratch_types=[
        pltpu.VMEM((CH,), jnp.int32),
        pltpu.VMEM((CH,), jnp.int32),
        pltpu.VMEM((CH,), jnp.int32),
        pltpu.VMEM((CH,), jnp.int32),
        pltpu.VMEM((L,), jnp.int32),
        pltpu.VMEM((CH,), jnp.int32),
        pltpu.VMEM((CH,), jnp.int32),
        pltpu.VMEM((CH,), jnp.int32),
        pltpu.VMEM((CH,), jnp.int32),
        pltpu.VMEM((CH,), jnp.int32),
        pltpu.VMEM((CH,), jnp.int32),
        pltpu.VMEM((CH,), jnp.int32),
        pltpu.VMEM((CH,), jnp.int32),
        pltpu.VMEM((CH, D), jnp.float32),
        pltpu.VMEM((CH, D), jnp.float32),
        pltpu.VMEM((CH, D), jnp.float32),
        pltpu.VMEM((CH, D), jnp.float32),
        pltpu.VMEM_SHARED((NP, D), jnp.float32),
        pltpu.SemaphoreType.DMA,
        pltpu.SemaphoreType.DMA,
        pltpu.SemaphoreType.DMA,
        pltpu.SemaphoreType.DMA,
        pltpu.SemaphoreType.DMA,
        pltpu.SemaphoreType.DMA,
        pltpu.SemaphoreType.DMA,
        pltpu.SemaphoreType.DMA,
        pltpu.SemaphoreType.DMA,
        pltpu.SemaphoreType.DMA,
        pltpu.SemaphoreType.DMA,
        pltpu.SemaphoreType.DMA,
    ],
)
def _sc_agg(u_hbm, pk_hbm, et1_hbm, agg_hbm,
            pk0, pk1, pk2, pk3, tvec,
            src0, src1, src2, src3, dst0, dst1, dst2, dst3,
            rows0, rows1, rows2, rows3, acc_sh,
            sem0, sem1, sem2, sem3, ssem0, ssem1, ssem2, ssem3,
            psem0, psem1, psem2, psem3):
    c = lax.axis_index("c")
    s = lax.axis_index("s")
    wid = s * NC + c
    ebase = wid * EPW

    zero = jnp.zeros((L,), jnp.float32)

    def zrow(i, _):
        for j in range(D // L):
            rows0[i, pl.ds(j * L, L)] = zero
        return ()

    lax.fori_loop(0, CH, zrow, ())

    def zcopy(k, _):
        pltpu.sync_copy(rows0, acc_sh.at[pl.ds(s * RPT + k * CH, CH)])
        return ()

    lax.fori_loop(0, RPT // CH, zcopy, ())
    plsc.subcore_barrier()

    pltpu.sync_copy(et1_hbm.at[pl.ds(E - L, L)], tvec)
    at_vec = tvec[...].astype(jnp.float32) + 1.0
    inv_at = (1.0 / at_vec)[L - 1]

    bufs = (
        (rows0, src0, dst0, pk0, sem0, ssem0, psem0),
        (rows1, src1, dst1, pk1, sem1, ssem1, psem1),
        (rows2, src2, dst2, pk2, sem2, ssem2, psem2),
        (rows3, src3, dst3, pk3, sem3, ssem3, psem3),
    )
    NB = len(bufs)

    def load_pk(k, buf):
        pltpu.async_copy(pk_hbm.at[pl.ds(ebase + k * CH, CH)], buf[3], buf[6])

    def issue(k, rows_b, src_b, dst_b, pk_b, sem_b, ssem_b, psem_b):
        pltpu.make_async_copy(
            pk_hbm.at[pl.ds(ebase + k * CH, CH)], pk_b, psem_b).wait()
        for j in range(CH // L):
            p = pk_b[pl.ds(j * L, L)]
            src_b[pl.ds(j * L, L)] = p & 0x3FFF
        pltpu.async_copy(u_hbm.at[src_b], rows_b, sem_b)

    def drain_scatter(rows_b, src_b, dst_b, pk_b, sem_b, ssem_b, psem_b):
        pltpu.make_async_copy(rows_b, acc_sh.at[dst_b], ssem_b).wait()

    def step(k, b):
        rows_b, src_b, dst_b, pk_b, sem_b, ssem_b, psem_b = bufs[b]
        pltpu.make_async_copy(u_hbm.at[src_b], rows_b, sem_b).wait()
        for j in range(CH // L):
            p = pk_b[pl.ds(j * L, L)]
            dst_b[pl.ds(j * L, L)] = lax.shift_right_logical(p, 14) & 0x3FFF
            t16 = lax.shift_right_logical(p, 28)
            scale = (t16.astype(jnp.float32) + 1.0) * inv_at
            rid = lax.iota(jnp.int32, L) + (j * L)
            cid = jnp.full((L,), D - 1, jnp.int32)
            plsc.store_scatter(rows_b, (rid, cid), scale)

        pltpu.async_copy(rows_b, acc_sh.at[dst_b], ssem_b, add=True)

        @pl.when(k + 4 < NCHUNK)
        def _():
            load_pk(k + 4, bufs[b])

        @pl.when(k >= 1)
        def _():
            drain_scatter(*bufs[(b + 3) % NB])

        @pl.when(k + 2 < NCHUNK)
        def _():
            issue(k + 2, *bufs[(b + 2) % NB])

    for k in range(min(4, NCHUNK)):
        load_pk(k, bufs[k % NB])
    issue(0, *bufs[0])
    issue(1, *bufs[1])

    def quad(i, _):
        for b in range(NB):
            step(NB * i + b, b)
        return ()

    nfull = (NCHUNK - 2) // NB
    lax.fori_loop(0, nfull, quad, ())
    for k in range(NB * nfull, NCHUNK):
        step(k, k % NB)
    drain_scatter(*bufs[(NCHUNK - 1) % NB])
    plsc.subcore_barrier()

    def ocopy(k, _):
        off = s * RPT + k * CH
        pltpu.sync_copy(acc_sh.at[pl.ds(off, CH)], agg_hbm.at[c, pl.ds(off, CH)])
        return ()

    lax.fori_loop(0, RPT // CH, ocopy, ())


def _fin_body(x_ref, a_ref, w_ref, o_ref):
    feat = x_ref[...] + a_ref[0] + a_ref[1]
    prod = lax.dot_general(
        feat, w_ref[...], (((1,), (1,)), ((), ())),
        preferred_element_type=jnp.float32,
    )
    o_ref[...] = jnp.tanh(prod)


def _finalize(raw, parts, W):
    return pl.pallas_call(
        _fin_body,
        grid=(10,),
        in_specs=[
            pl.BlockSpec((1000, D), lambda i: (i, 0)),
            pl.BlockSpec((NC, 1000, D), lambda i: (0, i, 0)),
            pl.BlockSpec((D, D), lambda i: (0, 0)),
        ],
        out_specs=pl.BlockSpec((1000, D), lambda i: (i, 0)),
        out_shape=jax.ShapeDtypeStruct((N, D), jnp.float32),
    )(raw, parts, W)


def kernel(raw_features, edge_index, edge_time, W):
    u = _normalize(raw_features)
    packed = edge_index[0] | (edge_index[1] << 14) | (edge_time << 28)
    parts = _sc_agg(u, packed, edge_time)
    return _finalize(raw_features, parts, W)


# speedup vs baseline: 1.1901x; 1.0866x over previous
import functools

import jax
import jax.numpy as jnp
from jax import lax
from jax.experimental import pallas as pl
from jax.experimental.pallas import tpu as pltpu
from jax.experimental.pallas import tpu_sc as plsc

N = 10000
D = 128
E = 320000
NC, NS, L = 2, 16, 16
NW = NC * NS
EPW = E // NW
CH = 80
NCHUNK = EPW // CH
NP = 10240
RPT = NP // NS


def _pre_body(x_ref, ei_ref, et_ref, o_ref, pk_ref):
    x = x_ref[...]
    ss = jnp.sum(x * x, axis=1, keepdims=True)
    nrm = jnp.maximum(jnp.sqrt(ss), 1e-12)
    o_ref[...] = x / nrm
    src = ei_ref[0]
    dst = ei_ref[1]
    t = et_ref[0]
    pk_ref[...] = src | (dst << 14) | (t << 28)


def _pre(raw, edge_index, edge_time):
    ER = E // 128
    u, pk = pl.pallas_call(
        _pre_body,
        out_shape=[
            jax.ShapeDtypeStruct((N, D), jnp.float32),
            jax.ShapeDtypeStruct((ER, 128), jnp.int32),
        ],
    )(raw, edge_index.reshape(2, ER, 128), edge_time.reshape(1, ER, 128))
    return u, pk.reshape(E)


_MESH = plsc.VectorSubcoreMesh(
    core_axis_name="c", subcore_axis_name="s", num_cores=NC, num_subcores=NS
)


@functools.partial(
    pl.kernel,
    out_type=jax.ShapeDtypeStruct((NC, NP, D), jnp.float32),
    mesh=_MESH,
    compiler_params=pltpu.CompilerParams(needs_layout_passes=False),
    scratch_types=[
        pltpu.VMEM((CH,), jnp.int32),
        pltpu.VMEM((CH,), jnp.int32),
        pltpu.VMEM((CH,), jnp.int32),
        pltpu.VMEM((CH,), jnp.int32),
        pltpu.VMEM((L,), jnp.int32),
        pltpu.VMEM((CH,), jnp.int32),
        pltpu.VMEM((CH,), jnp.int32),
        pltpu.VMEM((CH,), jnp.int32),
        pltpu.VMEM((CH,), jnp.int32),
        pltpu.VMEM((CH,), jnp.int32),
        pltpu.VMEM((CH,), jnp.int32),
        pltpu.VMEM((CH,), jnp.int32),
        pltpu.VMEM((CH,), jnp.int32),
        pltpu.VMEM((CH, D), jnp.float32),
        pltpu.VMEM((CH, D), jnp.float32),
        pltpu.VMEM((CH, D), jnp.float32),
        pltpu.VMEM((CH, D), jnp.float32),
        pltpu.VMEM_SHARED((NP, D), jnp.float32),
        pltpu.SemaphoreType.DMA,
        pltpu.SemaphoreType.DMA,
        pltpu.SemaphoreType.DMA,
        pltpu.SemaphoreType.DMA,
        pltpu.SemaphoreType.DMA,
        pltpu.SemaphoreType.DMA,
        pltpu.SemaphoreType.DMA,
        pltpu.SemaphoreType.DMA,
        pltpu.SemaphoreType.DMA,
        pltpu.SemaphoreType.DMA,
        pltpu.SemaphoreType.DMA,
        pltpu.SemaphoreType.DMA,
    ],
)
def _sc_agg(u_hbm, pk_hbm, et1_hbm, agg_hbm,
            pk0, pk1, pk2, pk3, tvec,
            src0, src1, src2, src3, dst0, dst1, dst2, dst3,
            rows0, rows1, rows2, rows3, acc_sh,
            sem0, sem1, sem2, sem3, ssem0, ssem1, ssem2, ssem3,
            psem0, psem1, psem2, psem3):
    c = lax.axis_index("c")
    s = lax.axis_index("s")
    wid = s * NC + c
    ebase = wid * EPW

    zero = jnp.zeros((L,), jnp.float32)

    def zrow(i, _):
        for j in range(D // L):
            rows0[i, pl.ds(j * L, L)] = zero
        return ()

    lax.fori_loop(0, CH, zrow, ())

    def zcopy(k, _):
        pltpu.sync_copy(rows0, acc_sh.at[pl.ds(s * RPT + k * CH, CH)])
        return ()

    lax.fori_loop(0, RPT // CH, zcopy, ())
    plsc.subcore_barrier()

    pltpu.sync_copy(et1_hbm.at[pl.ds(E - L, L)], tvec)
    at_vec = tvec[...].astype(jnp.float32) + 1.0
    inv_at = (1.0 / at_vec)[L - 1]

    bufs = (
        (rows0, src0, dst0, pk0, sem0, ssem0, psem0),
        (rows1, src1, dst1, pk1, sem1, ssem1, psem1),
        (rows2, src2, dst2, pk2, sem2, ssem2, psem2),
        (rows3, src3, dst3, pk3, sem3, ssem3, psem3),
    )
    NB = len(bufs)

    def load_pk(k, buf):
        pltpu.async_copy(pk_hbm.at[pl.ds(ebase + k * CH, CH)], buf[3], buf[6])

    def issue(k, rows_b, src_b, dst_b, pk_b, sem_b, ssem_b, psem_b):
        pltpu.make_async_copy(
            pk_hbm.at[pl.ds(ebase + k * CH, CH)], pk_b, psem_b).wait()
        for j in range(CH // L):
            p = pk_b[pl.ds(j * L, L)]
            src_b[pl.ds(j * L, L)] = p & 0x3FFF
        pltpu.async_copy(u_hbm.at[src_b], rows_b, sem_b)

    def drain_scatter(rows_b, src_b, dst_b, pk_b, sem_b, ssem_b, psem_b):
        pltpu.make_async_copy(rows_b, acc_sh.at[dst_b], ssem_b).wait()

    def step(k, b):
        rows_b, src_b, dst_b, pk_b, sem_b, ssem_b, psem_b = bufs[b]
        pltpu.make_async_copy(u_hbm.at[src_b], rows_b, sem_b).wait()
        for j in range(CH // L):
            p = pk_b[pl.ds(j * L, L)]
            dst_b[pl.ds(j * L, L)] = lax.shift_right_logical(p, 14) & 0x3FFF
            t16 = lax.shift_right_logical(p, 28)
            scale = (t16.astype(jnp.float32) + 1.0) * inv_at
            rid = lax.iota(jnp.int32, L) + (j * L)
            cid = jnp.full((L,), D - 1, jnp.int32)
            plsc.store_scatter(rows_b, (rid, cid), scale)

        pltpu.async_copy(rows_b, acc_sh.at[dst_b], ssem_b, add=True)

        @pl.when(k + 4 < NCHUNK)
        def _():
            load_pk(k + 4, bufs[b])

        @pl.when(k >= 1)
        def _():
            drain_scatter(*bufs[(b + 3) % NB])

        @pl.when(k + 2 < NCHUNK)
        def _():
            issue(k + 2, *bufs[(b + 2) % NB])

    for k in range(min(4, NCHUNK)):
        load_pk(k, bufs[k % NB])
    issue(0, *bufs[0])
    issue(1, *bufs[1])

    def quad(i, _):
        for b in range(NB):
            step(NB * i + b, b)
        return ()

    nfull = (NCHUNK - 2) // NB
    lax.fori_loop(0, nfull, quad, ())
    for k in range(NB * nfull, NCHUNK):
        step(k, k % NB)
    drain_scatter(*bufs[(NCHUNK - 1) % NB])
    plsc.subcore_barrier()

    def ocopy(k, _):
        off = s * RPT + k * CH
        pltpu.sync_copy(acc_sh.at[pl.ds(off, CH)], agg_hbm.at[c, pl.ds(off, CH)])
        return ()

    lax.fori_loop(0, RPT // CH, ocopy, ())


def _fin_body(x_ref, a_ref, w_ref, o_ref):
    feat = x_ref[...] + a_ref[0] + a_ref[1]
    prod = lax.dot_general(
        feat, w_ref[...], (((1,), (1,)), ((), ())),
        preferred_element_type=jnp.float32,
    )
    o_ref[...] = jnp.tanh(prod)


def _finalize(raw, parts, W):
    return pl.pallas_call(
        _fin_body,
        grid=(10,),
        in_specs=[
            pl.BlockSpec((1000, D), lambda i: (i, 0)),
            pl.BlockSpec((NC, 1000, D), lambda i: (0, i, 0)),
            pl.BlockSpec((D, D), lambda i: (0, 0)),
        ],
        out_specs=pl.BlockSpec((1000, D), lambda i: (i, 0)),
        out_shape=jax.ShapeDtypeStruct((N, D), jnp.float32),
    )(raw, parts, W)


def kernel(raw_features, edge_index, edge_time, W):
    u, packed = _pre(raw_features, edge_index, edge_time)
    parts = _sc_agg(u, packed, edge_time)
    return _finalize(raw_features, parts, W)
